# baseline (speedup 1.0000x reference)
import functools
import jax
import jax.numpy as jnp
from jax import lax
from jax.experimental import pallas as pl
from jax.experimental.pallas import tpu as pltpu
from jax.experimental.pallas import tpu_sc as plsc


def _make_gather(B, D, NW, CHUNK):
    b_per_w = B // NW
    n_chunks = b_per_w // CHUNK
    mesh = plsc.VectorSubcoreMesh(core_axis_name="c", subcore_axis_name="s")

    @functools.partial(
        pl.kernel,
        mesh=mesh,
        compiler_params=pltpu.CompilerParams(use_tc_tiling_on_sc=False),
        out_type=[
            jax.ShapeDtypeStruct((B, D), jnp.float32),
            jax.ShapeDtypeStruct((B, D), jnp.float32),
        ],
        scratch_types=[
            pltpu.VMEM((n_chunks, CHUNK), jnp.int32),
            pltpu.VMEM((n_chunks, CHUNK), jnp.int32),
            pltpu.VMEM((n_chunks, CHUNK, D), jnp.float32),
            pltpu.VMEM((n_chunks, CHUNK, D), jnp.float32),
            pltpu.SemaphoreType.DMA,
            pltpu.SemaphoreType.DMA,
        ],
    )
    def k(iw_hbm, cw_hbm, tt_hbm, ct_hbm, out_i_hbm, out_c_hbm,
          idx_i, idx_c, rows_i, rows_c, sem_i, sem_c):
        nc = jax.lax.axis_size("c")
        wid = lax.axis_index("s") * nc + lax.axis_index("c")
        base = wid * b_per_w
        for j in range(n_chunks):
            pltpu.sync_copy(iw_hbm.at[pl.ds(base + j * CHUNK, CHUNK)],
                            idx_i.at[j])
            pltpu.sync_copy(cw_hbm.at[pl.ds(base + j * CHUNK, CHUNK)],
                            idx_c.at[j])
        copies = []
        for j in range(n_chunks):
            copies.append(
                pltpu.async_copy(tt_hbm.at[idx_i.at[j]], rows_i.at[j], sem_i))
            copies.append(
                pltpu.async_copy(ct_hbm.at[idx_c.at[j]], rows_c.at[j], sem_c))
        for cp in copies:
            cp.wait()
        for j in range(n_chunks):
            pltpu.sync_copy(rows_i.at[j],
                            out_i_hbm.at[pl.ds(base + j * CHUNK, CHUNK)])
            pltpu.sync_copy(rows_c.at[j],
                            out_c_hbm.at[pl.ds(base + j * CHUNK, CHUNK)])

    return k


def kernel(input_word, context_word, target_table, context_table):
    B = input_word.shape[0]
    D = target_table.shape[1]
    gather = _make_gather(B, D, NW=32, CHUNK=128)
    out_i, out_c = gather(
        input_word.astype(jnp.int32),
        context_word.astype(jnp.int32),
        target_table,
        context_table,
    )
    return (out_i, out_c)


# speedup vs baseline: 1.4540x; 1.4540x over previous
import functools
import jax
import jax.numpy as jnp
from jax import lax
from jax.experimental import pallas as pl
from jax.experimental.pallas import tpu as pltpu
from jax.experimental.pallas import tpu_sc as plsc


def _make_gather(B, D, NW, G):
    b_per_w = B // NW
    n_g = b_per_w // G

    mesh = plsc.VectorSubcoreMesh(core_axis_name="c", subcore_axis_name="s")

    @functools.partial(
        pl.kernel,
        mesh=mesh,
        out_type=[
            jax.ShapeDtypeStruct((B, D), jnp.float32),
            jax.ShapeDtypeStruct((B, D), jnp.float32),
        ],
        scratch_types=[
            pltpu.VMEM((n_g, G), jnp.int32),
            pltpu.VMEM((2, G, D), jnp.float32),
            pltpu.SemaphoreType.DMA,
            pltpu.SemaphoreType.DMA,
            pltpu.SemaphoreType.DMA,
            pltpu.SemaphoreType.DMA,
        ],
    )
    def k(iw_hbm, cw_hbm, t_hbm, c_hbm, out_i_hbm, out_c_hbm,
          idx_v, rows_v, sem_g0, sem_g1, sem_o0, sem_o1):
        nc = plsc.get_sparse_core_info().num_cores
        wid = lax.axis_index("s") * nc + lax.axis_index("c")
        base = wid * b_per_w
        sem_g = (sem_g0, sem_g1)
        sem_o = (sem_o0, sem_o1)

        for idx_hbm, src_hbm, out_hbm in (
                (iw_hbm, t_hbm, out_i_hbm),
                (cw_hbm, c_hbm, out_c_hbm)):
            for g in range(n_g):
                pltpu.sync_copy(idx_hbm.at[pl.ds(base + g * G, G)],
                                idx_v.at[g])

            def fire(g, p):
                v16 = idx_v[g, pl.ds(0, 16)]
                cps = []
                for j in range(G):
                    r = v16[j] if j < 16 else idx_v[g, pl.ds(16, 16)][j - 16]
                    cps.append(pltpu.async_copy(
                        src_hbm.at[r], rows_v.at[p, j], sem_g[p]))
                return cps

            out_copies = [None, None]
            copies = [None, None]
            copies[0] = fire(0, 0)
            for g in range(n_g):
                p = g & 1
                for cp in copies[p]:
                    cp.wait()
                if g + 1 < n_g:
                    copies[1 - p] = fire(g + 1, 1 - p)
                if out_copies[p] is not None:
                    out_copies[p].wait()
                out_copies[p] = pltpu.async_copy(
                    rows_v.at[p],
                    out_hbm.at[pl.ds(base + g * G, G)],
                    sem_o[p])
            for oc in out_copies:
                if oc is not None:
                    oc.wait()

    return k


def kernel(input_word, context_word, target_table, context_table):
    V, D = target_table.shape
    B = input_word.shape[0]
    gather = _make_gather(B, D, NW=32, G=16)
    out_i, out_c = gather(
        input_word.astype(jnp.int32),
        context_word.astype(jnp.int32),
        target_table,
        context_table,
    )
    return (out_i, out_c)
